# baseline (speedup 1.0000x reference)
import jax
import jax.numpy as jnp
from jax import lax
from jax.experimental import pallas as pl
from jax.experimental.pallas import tpu as pltpu
from jax.experimental.pallas import tpu_sc as plsc

N = 100000
E = 6400000
H = 32
HH = H // 2

NC = 2
NS = 16

NPAD = 100352
ZROWS = NPAD // NS

EROWS = E // 128
EROWSP = 50176
TROWS = EROWSP // NS
IDXB = 16
BLOCKS = TROWS // IDXB
CR = 4
CPB = IDXB // CR

CROWS = EROWSP // NC
CTROWS = CROWS // NS
CBLOCKS = CTROWS // IDXB

BLK = 3584


def _sc_segsum_body(y2_hbm, src2_hbm, dst_hbm, z_hbm, agg_hbm,
                    s0_v, d0_v, s1_v, d1_v, rows0_v, rows1_v,
                    acc_sh, sem0, sem1, sem2, sem3):
    cid = lax.axis_index("c")
    sid = lax.axis_index("s")

    pltpu.sync_copy(z_hbm, acc_sh.at[pl.ds(sid * ZROWS, ZROWS)])
    plsc.subcore_barrier()

    src_base = (cid * EROWSP + sid * TROWS) * 128
    dst_base = sid * TROWS * 128
    BE = IDXB * 128
    CE = CR * 128

    bufs = (rows0_v, rows1_v)
    gsems = (sem0, sem1)
    ssems = (sem2, sem3)

    def fire(sv, c, p):
        return pltpu.async_copy(y2_hbm.at[sv.at[pl.ds(c * CE, CE)]],
                                bufs[p], gsems[p])

    def scat(dv, c, p):
        return pltpu.async_copy(bufs[p],
                                acc_sh.at[dv.at[pl.ds(c * CE, CE)]],
                                ssems[p], add=True)

    pltpu.sync_copy(src2_hbm.at[pl.ds(src_base, BE)], s0_v)
    pltpu.sync_copy(dst_hbm.at[pl.ds(dst_base, BE)], d0_v)

    def pair(i, carry):
        row_b = (2 * i + 1) * BE
        row_n = lax.min((2 * i + 2) * BE, (TROWS - IDXB) * 128)
        g = fire(s0_v, 0, 0)
        pltpu.sync_copy(src2_hbm.at[pl.ds(src_base + row_b, BE)], s1_v)
        pltpu.sync_copy(dst_hbm.at[pl.ds(dst_base + row_b, BE)], d1_v)
        sc = [None, None]
        for c in range(CPB):
            q = (c + 1) % 2
            if sc[q] is not None:
                sc[q].wait()
                sc[q] = None
            nxt = (fire(s0_v, c + 1, q) if c + 1 < CPB
                   else fire(s1_v, 0, q))
            g.wait()
            sc[c % 2] = scat(d0_v, c, c % 2)
            g = nxt
        sc[1].wait()
        sc[1] = None
        pltpu.sync_copy(src2_hbm.at[pl.ds(src_base + row_n, BE)], s0_v)
        pltpu.sync_copy(dst_hbm.at[pl.ds(dst_base + row_n, BE)], d0_v)
        for c in range(CPB):
            q = (c + 1) % 2
            if sc[q] is not None:
                sc[q].wait()
                sc[q] = None
            nxt = fire(s1_v, c + 1, q) if c + 1 < CPB else None
            g.wait()
            sc[c % 2] = scat(d1_v, c, c % 2)
            g = nxt
        sc[1].wait()
        return carry

    lax.fori_loop(0, BLOCKS // 2, pair, 0)
    plsc.subcore_barrier()

    pltpu.sync_copy(acc_sh.at[pl.ds(sid * ZROWS, ZROWS)],
                    agg_hbm.at[pl.ds(cid * NPAD + sid * ZROWS, ZROWS)])


def _sc_count_body(dst_hbm, ones_hbm, z_hbm, cnt_hbm,
                   dst_v, ones_v, acc_sh):
    cid = lax.axis_index("c")
    sid = lax.axis_index("s")

    pltpu.sync_copy(ones_hbm, ones_v)
    pltpu.sync_copy(z_hbm, acc_sh.at[pl.ds(sid * ZROWS, ZROWS)])
    plsc.subcore_barrier()

    base = cid * CROWS + sid * CTROWS

    def block(b, carry):
        pltpu.sync_copy(dst_hbm.at[pl.ds(base + b * IDXB, IDXB)], dst_v)
        for j in range(IDXB):
            pltpu.sync_copy(ones_v, acc_sh.at[dst_v.at[j]], add=True)
        return carry

    lax.fori_loop(0, CBLOCKS, block, 0)
    plsc.subcore_barrier()

    pltpu.sync_copy(acc_sh.at[pl.ds(sid * ZROWS, ZROWS)],
                    cnt_hbm.at[pl.ds(cid * NPAD + sid * ZROWS, ZROWS)])


def _sc_segsum(y2, src2, dst2d, zeros16, mesh):
    return pl.kernel(
        _sc_segsum_body,
        out_type=jax.ShapeDtypeStruct((NC * NPAD, HH), jnp.float32),
        mesh=mesh,
        compiler_params=pltpu.CompilerParams(use_tc_tiling_on_sc=False),
        scratch_types=[
            pltpu.VMEM((IDXB * 128,), jnp.int32),
            pltpu.VMEM((IDXB * 128,), jnp.int32),
            pltpu.VMEM((IDXB * 128,), jnp.int32),
            pltpu.VMEM((IDXB * 128,), jnp.int32),
            pltpu.VMEM((CR * 128, HH), jnp.float32),
            pltpu.VMEM((CR * 128, HH), jnp.float32),
            pltpu.VMEM_SHARED((NPAD, HH), jnp.float32),
            pltpu.SemaphoreType.DMA,
            pltpu.SemaphoreType.DMA,
            pltpu.SemaphoreType.DMA,
            pltpu.SemaphoreType.DMA,
        ],
    )(y2, src2, dst2d, zeros16)


def _sc_count(dst2d, ones8, zeros8, mesh):
    return pl.kernel(
        _sc_count_body,
        out_type=jax.ShapeDtypeStruct((NC * NPAD, 8), jnp.float32),
        mesh=mesh,
        compiler_params=pltpu.CompilerParams(use_tc_tiling_on_sc=False),
        scratch_types=[
            pltpu.VMEM((IDXB, 128), jnp.int32),
            pltpu.VMEM((128, 8), jnp.float32),
            pltpu.VMEM_SHARED((NPAD, 8), jnp.float32),
        ],
    )(dst2d, ones8, zeros8)


def _tc_pre_body(x_ref, wlt_ref, y2_ref):
    y = jnp.dot(x_ref[...], wlt_ref[...], preferred_element_type=jnp.float32)
    y2_ref[0] = y[:, :HH]
    y2_ref[1] = y[:, HH:]


def _tc_mid_body(agg0_ref, agg1_ref, h_ref, cnt0_ref, cnt1_ref, bl_ref,
                 wrt_ref, wlt_ref, hout_ref, y2_ref):
    cnt = cnt0_ref[:, 0:1] + cnt1_ref[:, 0:1]
    recip = 1.0 / jnp.maximum(cnt, 1.0)
    mean = jnp.concatenate([agg0_ref[...], agg1_ref[...]], axis=1) * recip
    hn = (mean + bl_ref[...] +
          jnp.dot(h_ref[...], wrt_ref[...], preferred_element_type=jnp.float32))
    hn = jnp.maximum(hn, 0.0)
    hout_ref[...] = hn
    y = jnp.dot(hn, wlt_ref[...], preferred_element_type=jnp.float32)
    y2_ref[0] = y[:, :HH]
    y2_ref[1] = y[:, HH:]


def _tc_final_body(agg0_ref, agg1_ref, h_ref, cnt0_ref, cnt1_ref, bl_ref,
                   wrt_ref, wp1t_ref, bp1_ref, wp2t_ref, bp2_ref, out_ref):
    cnt = cnt0_ref[:, 0:1] + cnt1_ref[:, 0:1]
    recip = 1.0 / jnp.maximum(cnt, 1.0)
    mean = jnp.concatenate([agg0_ref[...], agg1_ref[...]], axis=1) * recip
    hn = (mean + bl_ref[...] +
          jnp.dot(h_ref[...], wrt_ref[...], preferred_element_type=jnp.float32))
    z = jnp.maximum(
        jnp.dot(hn, wp1t_ref[...], preferred_element_type=jnp.float32)
        + bp1_ref[...], 0.0)
    out_ref[...] = (jnp.dot(z, wp2t_ref[...], preferred_element_type=jnp.float32)
                    + bp2_ref[...])


def _row_spec(width):
    return pl.BlockSpec((BLK, width), lambda i: (i, 0))


def _full_spec(shape):
    return pl.BlockSpec(shape, lambda i: (0,) * len(shape))


def _y2_spec():
    return pl.BlockSpec((2, BLK, HH), lambda i: (0, i, 0))


def _tc_pre(x, wlt):
    d_in = x.shape[1]
    return pl.pallas_call(
        _tc_pre_body,
        grid=(NPAD // BLK,),
        in_specs=[_row_spec(d_in), _full_spec(wlt.shape)],
        out_specs=_y2_spec(),
        out_shape=jax.ShapeDtypeStruct((2, NPAD, HH), jnp.float32),
    )(x, wlt)


def _tc_mid(agg0, agg1, h, cnt0, cnt1, bl, wrt, wlt):
    d_in = h.shape[1]
    return pl.pallas_call(
        _tc_mid_body,
        grid=(NPAD // BLK,),
        in_specs=[_row_spec(HH), _row_spec(HH), _row_spec(d_in),
                  _row_spec(8), _row_spec(8),
                  _full_spec((1, H)), _full_spec(wrt.shape),
                  _full_spec((H, H))],
        out_specs=(_row_spec(H), _y2_spec()),
        out_shape=(jax.ShapeDtypeStruct((NPAD, H), jnp.float32),
                   jax.ShapeDtypeStruct((2, NPAD, HH), jnp.float32)),
    )(agg0, agg1, h, cnt0, cnt1, bl, wrt, wlt)


def _tc_final(agg0, agg1, h, cnt0, cnt1, bl, wrt, wp1t, bp1, wp2t, bp2):
    return pl.pallas_call(
        _tc_final_body,
        grid=(NPAD // BLK,),
        in_specs=[_row_spec(HH), _row_spec(HH), _row_spec(H),
                  _row_spec(8), _row_spec(8),
                  _full_spec((1, H)), _full_spec((H, H)),
                  _full_spec((H, H)), _full_spec((1, H)),
                  _full_spec((H, 1)), _full_spec((1, 1))],
        out_specs=_row_spec(1),
        out_shape=jax.ShapeDtypeStruct((NPAD, 1), jnp.float32),
    )(agg0, agg1, h, cnt0, cnt1, bl, wrt, wp1t, bp1, wp2t, bp2)


def kernel(x, edge_index, batch, params):
    del batch
    mesh = plsc.VectorSubcoreMesh(core_axis_name="c", subcore_axis_name="s",
                                  num_cores=NC, num_subcores=NS)

    pad_rows = EROWSP - EROWS
    src2d = jnp.pad(edge_index[0].reshape(EROWS, 128), ((0, pad_rows), (0, 0)))
    dst2d = jnp.pad(edge_index[1].reshape(EROWS, 128), ((0, pad_rows), (0, 0)),
                    constant_values=N)
    src2 = jnp.concatenate([src2d, src2d + NPAD], axis=0).reshape(-1)
    dst_flat = dst2d.reshape(-1)
    xp = jnp.pad(x, ((0, NPAD - N), (0, 0)))
    zeros16 = jnp.zeros((ZROWS, HH), jnp.float32)
    zeros8 = jnp.zeros((ZROWS, 8), jnp.float32)
    ones8 = jnp.ones((128, 8), jnp.float32)

    cnt = _sc_count(dst2d, ones8, zeros8, mesh)
    cnt0 = cnt[:NPAD]
    cnt1 = cnt[NPAD:]

    L = 5
    h = xp
    y2 = _tc_pre(xp, params["Wl0"].T)
    out = None
    for i in range(L):
        agg = _sc_segsum(y2.reshape(NC * NPAD, HH), src2, dst_flat,
                         zeros16, mesh)
        agg0 = agg[:NPAD]
        agg1 = agg[NPAD:]
        bl = params[f"bl{i}"].reshape(1, H)
        wrt = params[f"Wr{i}"].T
        if i < L - 1:
            h, y2 = _tc_mid(agg0, agg1, h, cnt0, cnt1, bl, wrt,
                            params[f"Wl{i+1}"].T)
        else:
            out = _tc_final(agg0, agg1, h, cnt0, cnt1, bl, wrt,
                            params["Wp1"].T, params["bp1"].reshape(1, H),
                            params["Wp2"].T, params["bp2"].reshape(1, 1))
    return out[:N]


# speedup vs baseline: 1.0211x; 1.0211x over previous
import jax
import jax.numpy as jnp
from jax import lax
from jax.experimental import pallas as pl
from jax.experimental.pallas import tpu as pltpu
from jax.experimental.pallas import tpu_sc as plsc

N = 100000
E = 6400000
H = 32
HH = H // 2

NC = 2
NS = 16

NPAD = 100352
ZROWS = NPAD // NS

EROWS = E // 128
EROWSP = 50176
TROWS = EROWSP // NS
IDXB = 16
BLOCKS = TROWS // IDXB
CR = 4
CPB = IDXB // CR

CROWS = EROWSP // NC
CTROWS = CROWS // NS
CBLOCKS = CTROWS // IDXB

BLK = 3584


def _sc_segsum_body(y2_hbm, src2_hbm, dst_hbm, z_hbm, agg_hbm,
                    s0_v, d0_v, s1_v, d1_v, rows0_v, rows1_v,
                    acc_sh, sem0, sem1, sem2, sem3):
    cid = lax.axis_index("c")
    sid = lax.axis_index("s")

    pltpu.sync_copy(z_hbm, acc_sh.at[pl.ds(sid * ZROWS, ZROWS)])
    plsc.subcore_barrier()

    src_base = (cid * EROWSP + sid * TROWS) * 128
    dst_base = sid * TROWS * 128
    BE = IDXB * 128
    CE = CR * 128

    bufs = (rows0_v, rows1_v)
    gsems = (sem0, sem1)
    ssems = (sem2, sem3)

    def fire(sv, c, p):
        return pltpu.async_copy(y2_hbm.at[sv.at[pl.ds(c * CE, CE)]],
                                bufs[p], gsems[p])

    def scat(dv, c, p):
        return pltpu.async_copy(bufs[p],
                                acc_sh.at[dv.at[pl.ds(c * CE, CE)]],
                                ssems[p], add=True)

    pltpu.sync_copy(src2_hbm.at[pl.ds(src_base, BE)], s0_v)
    pltpu.sync_copy(dst_hbm.at[pl.ds(dst_base, BE)], d0_v)

    def pair(i, carry):
        row_b = (2 * i + 1) * BE
        row_n = lax.min((2 * i + 2) * BE, (TROWS - IDXB) * 128)
        g = fire(s0_v, 0, 0)
        pltpu.sync_copy(src2_hbm.at[pl.ds(src_base + row_b, BE)], s1_v)
        pltpu.sync_copy(dst_hbm.at[pl.ds(dst_base + row_b, BE)], d1_v)
        sc = [None, None]
        for c in range(CPB):
            q = (c + 1) % 2
            if sc[q] is not None:
                sc[q].wait()
                sc[q] = None
            nxt = (fire(s0_v, c + 1, q) if c + 1 < CPB
                   else fire(s1_v, 0, q))
            g.wait()
            sc[c % 2] = scat(d0_v, c, c % 2)
            g = nxt
        sc[1].wait()
        sc[1] = None
        pltpu.sync_copy(src2_hbm.at[pl.ds(src_base + row_n, BE)], s0_v)
        pltpu.sync_copy(dst_hbm.at[pl.ds(dst_base + row_n, BE)], d0_v)
        for c in range(CPB):
            q = (c + 1) % 2
            if sc[q] is not None:
                sc[q].wait()
                sc[q] = None
            nxt = fire(s1_v, c + 1, q) if c + 1 < CPB else None
            g.wait()
            sc[c % 2] = scat(d1_v, c, c % 2)
            g = nxt
        sc[1].wait()
        return carry

    lax.fori_loop(0, BLOCKS // 2, pair, 0)
    plsc.subcore_barrier()

    pltpu.sync_copy(acc_sh.at[pl.ds(sid * ZROWS, ZROWS)],
                    agg_hbm.at[pl.ds(cid * NPAD + sid * ZROWS, ZROWS)])


def _sc_count_body(dst_hbm, ones_hbm, z_hbm, cnt_hbm,
                   dst_v, ones_v, acc_sh):
    cid = lax.axis_index("c")
    sid = lax.axis_index("s")

    pltpu.sync_copy(ones_hbm, ones_v)
    pltpu.sync_copy(z_hbm, acc_sh.at[pl.ds(sid * ZROWS, ZROWS)])
    plsc.subcore_barrier()

    base = cid * CROWS + sid * CTROWS

    def block(b, carry):
        pltpu.sync_copy(dst_hbm.at[pl.ds(base + b * IDXB, IDXB)], dst_v)
        for j in range(IDXB):
            pltpu.sync_copy(ones_v, acc_sh.at[dst_v.at[j]], add=True)
        return carry

    lax.fori_loop(0, CBLOCKS, block, 0)
    plsc.subcore_barrier()

    pltpu.sync_copy(acc_sh.at[pl.ds(sid * ZROWS, ZROWS)],
                    cnt_hbm.at[pl.ds(cid * NPAD + sid * ZROWS, ZROWS)])


def _sc_segsum(y2, src2, dst2d, zeros16, mesh):
    return pl.kernel(
        _sc_segsum_body,
        out_type=jax.ShapeDtypeStruct((NC * NPAD, HH), jnp.float32),
        mesh=mesh,
        compiler_params=pltpu.CompilerParams(use_tc_tiling_on_sc=False),
        scratch_types=[
            pltpu.VMEM((IDXB * 128,), jnp.int32),
            pltpu.VMEM((IDXB * 128,), jnp.int32),
            pltpu.VMEM((IDXB * 128,), jnp.int32),
            pltpu.VMEM((IDXB * 128,), jnp.int32),
            pltpu.VMEM((CR * 128, HH), jnp.float32),
            pltpu.VMEM((CR * 128, HH), jnp.float32),
            pltpu.VMEM_SHARED((NPAD, HH), jnp.float32),
            pltpu.SemaphoreType.DMA,
            pltpu.SemaphoreType.DMA,
            pltpu.SemaphoreType.DMA,
            pltpu.SemaphoreType.DMA,
        ],
    )(y2, src2, dst2d, zeros16)


def _sc_count(dst2d, ones8, zeros8, mesh):
    return pl.kernel(
        _sc_count_body,
        out_type=jax.ShapeDtypeStruct((NC * NPAD, 8), jnp.float32),
        mesh=mesh,
        compiler_params=pltpu.CompilerParams(use_tc_tiling_on_sc=False),
        scratch_types=[
            pltpu.VMEM((IDXB, 128), jnp.int32),
            pltpu.VMEM((128, 8), jnp.float32),
            pltpu.VMEM_SHARED((NPAD, 8), jnp.float32),
        ],
    )(dst2d, ones8, zeros8)


def _tc_pre_body(x_ref, y2_ref):
    y2_ref[0] = x_ref[:, :HH]
    y2_ref[1] = x_ref[:, HH:]


def _tc_mid_body(agg0_ref, agg1_ref, h_ref, cnt0_ref, cnt1_ref, bl_ref,
                 wrt_ref, wlt_ref, hout_ref, y2_ref):
    cnt = cnt0_ref[:, 0:1] + cnt1_ref[:, 0:1]
    recip = 1.0 / jnp.maximum(cnt, 1.0)
    mean = jnp.concatenate([agg0_ref[...], agg1_ref[...]], axis=1) * recip
    hn = (jnp.dot(mean, wlt_ref[...], preferred_element_type=jnp.float32)
          + bl_ref[...] +
          jnp.dot(h_ref[...], wrt_ref[...], preferred_element_type=jnp.float32))
    hn = jnp.maximum(hn, 0.0)
    hout_ref[...] = hn
    y2_ref[0] = hn[:, :HH]
    y2_ref[1] = hn[:, HH:]


def _tc_final_body(agg0_ref, agg1_ref, h_ref, cnt0_ref, cnt1_ref, bl_ref,
                   wrt_ref, wlt_ref, wp1t_ref, bp1_ref, wp2t_ref, bp2_ref,
                   out_ref):
    cnt = cnt0_ref[:, 0:1] + cnt1_ref[:, 0:1]
    recip = 1.0 / jnp.maximum(cnt, 1.0)
    mean = jnp.concatenate([agg0_ref[...], agg1_ref[...]], axis=1) * recip
    hn = (jnp.dot(mean, wlt_ref[...], preferred_element_type=jnp.float32)
          + bl_ref[...] +
          jnp.dot(h_ref[...], wrt_ref[...], preferred_element_type=jnp.float32))
    z = jnp.maximum(
        jnp.dot(hn, wp1t_ref[...], preferred_element_type=jnp.float32)
        + bp1_ref[...], 0.0)
    out_ref[...] = (jnp.dot(z, wp2t_ref[...], preferred_element_type=jnp.float32)
                    + bp2_ref[...])


def _row_spec(width):
    return pl.BlockSpec((BLK, width), lambda i: (i, 0))


def _full_spec(shape):
    return pl.BlockSpec(shape, lambda i: (0,) * len(shape))


def _y2_spec():
    return pl.BlockSpec((2, BLK, HH), lambda i: (0, i, 0))


def _tc_pre(x):
    return pl.pallas_call(
        _tc_pre_body,
        grid=(NPAD // BLK,),
        in_specs=[_row_spec(H)],
        out_specs=_y2_spec(),
        out_shape=jax.ShapeDtypeStruct((2, NPAD, HH), jnp.float32),
    )(x)


def _tc_mid(agg0, agg1, h, cnt0, cnt1, bl, wrt, wlt):
    return pl.pallas_call(
        _tc_mid_body,
        grid=(NPAD // BLK,),
        in_specs=[_row_spec(HH), _row_spec(HH), _row_spec(H),
                  _row_spec(8), _row_spec(8),
                  _full_spec((1, H)), _full_spec((H, H)),
                  _full_spec((H, H))],
        out_specs=(_row_spec(H), _y2_spec()),
        out_shape=(jax.ShapeDtypeStruct((NPAD, H), jnp.float32),
                   jax.ShapeDtypeStruct((2, NPAD, HH), jnp.float32)),
    )(agg0, agg1, h, cnt0, cnt1, bl, wrt, wlt)


def _tc_final(agg0, agg1, h, cnt0, cnt1, bl, wrt, wlt, wp1t, bp1, wp2t, bp2):
    return pl.pallas_call(
        _tc_final_body,
        grid=(NPAD // BLK,),
        in_specs=[_row_spec(HH), _row_spec(HH), _row_spec(H),
                  _row_spec(8), _row_spec(8),
                  _full_spec((1, H)), _full_spec((H, H)),
                  _full_spec((H, H)), _full_spec((H, H)),
                  _full_spec((1, H)), _full_spec((H, 1)), _full_spec((1, 1))],
        out_specs=_row_spec(1),
        out_shape=jax.ShapeDtypeStruct((NPAD, 1), jnp.float32),
    )(agg0, agg1, h, cnt0, cnt1, bl, wrt, wlt, wp1t, bp1, wp2t, bp2)


def kernel(x, edge_index, batch, params):
    del batch
    mesh = plsc.VectorSubcoreMesh(core_axis_name="c", subcore_axis_name="s",
                                  num_cores=NC, num_subcores=NS)

    pad_rows = EROWSP - EROWS
    src2d = jnp.pad(edge_index[0].reshape(EROWS, 128), ((0, pad_rows), (0, 0)))
    dst2d = jnp.pad(edge_index[1].reshape(EROWS, 128), ((0, pad_rows), (0, 0)),
                    constant_values=N)
    src2 = jnp.concatenate([src2d, src2d + NPAD], axis=0).reshape(-1)
    dst_flat = dst2d.reshape(-1)
    xp = jnp.pad(x, ((0, NPAD - N), (0, H - x.shape[1])))
    zeros16 = jnp.zeros((ZROWS, HH), jnp.float32)
    zeros8 = jnp.zeros((ZROWS, 8), jnp.float32)
    ones8 = jnp.ones((128, 8), jnp.float32)

    cnt = _sc_count(dst2d, ones8, zeros8, mesh)
    cnt0 = cnt[:NPAD]
    cnt1 = cnt[NPAD:]

    L = 5
    h = xp
    h2 = _tc_pre(xp)
    out = None
    for i in range(L):
        agg = _sc_segsum(h2.reshape(NC * NPAD, HH), src2, dst_flat,
                         zeros16, mesh)
        agg0 = agg[:NPAD]
        agg1 = agg[NPAD:]
        bl = params[f"bl{i}"].reshape(1, H)
        wlt = params[f"Wl{i}"].T
        wrt = params[f"Wr{i}"].T
        if i == 0:
            wlt = jnp.pad(wlt, ((0, H - wlt.shape[0]), (0, 0)))
            wrt = jnp.pad(wrt, ((0, H - wrt.shape[0]), (0, 0)))
        if i < L - 1:
            h, h2 = _tc_mid(agg0, agg1, h, cnt0, cnt1, bl, wrt, wlt)
        else:
            out = _tc_final(agg0, agg1, h, cnt0, cnt1, bl, wrt, wlt,
                            params["Wp1"].T, params["bp1"].reshape(1, H),
                            params["Wp2"].T, params["bp2"].reshape(1, 1))
    return out[:N]
